# baseline (speedup 1.0000x reference)
import functools

import jax
import jax.numpy as jnp
from jax import lax
from jax.experimental import pallas as pl
from jax.experimental.pallas import tpu as pltpu
from jax.experimental.pallas import tpu_sc as plsc

N_NODES_ = 10000
D_ = 128
C_ = 8
E_ = 320000

_NC = 2
_NS = 16
_NW = _NC * _NS
_CHUNK = 128
_STEPS = 80
_EPAD = _NW * _STEPS * _CHUNK
_LAST_STEPS = 20
_NACC = N_NODES_
_COPY_ROWS = 624
_ZROWS = 16
_NBUF = 2


def _edge_partials(x, packed2):
    mesh = plsc.VectorSubcoreMesh(core_axis_name="c", subcore_axis_name="s")

    @functools.partial(
        pl.kernel,
        mesh=mesh,
        out_type=jax.ShapeDtypeStruct((_NC, _NACC, D_), jnp.float32),
        scratch_types=[
            pltpu.VMEM((_STEPS, _CHUNK), jnp.int32),
            pltpu.VMEM((_NBUF, _CHUNK), jnp.int32),
            pltpu.VMEM((_NBUF, _CHUNK), jnp.int32),
            pltpu.VMEM((_NBUF, _CHUNK, D_), jnp.float32),
            pltpu.VMEM((_ZROWS, D_), jnp.float32),
            pltpu.VMEM_SHARED((_NACC, D_), jnp.float32),
            pltpu.SemaphoreType.DMA,
            pltpu.SemaphoreType.DMA,
            pltpu.SemaphoreType.DMA,
            pltpu.SemaphoreType.DMA,
            pltpu.SemaphoreType.DMA,
        ],
    )
    def k(x_hbm, pk_hbm, out_hbm, pk_v, src_r, dst_r, rows_v, zbuf, acc,
          sem0, sem1, ss0, ss1, zsem):
        sems = (sem0, sem1)
        ssems = (ss0, ss1)
        c = lax.axis_index("c")
        s = lax.axis_index("s")
        wid = c * _NS + s
        n_steps = jnp.where(wid == _NW - 1, _LAST_STEPS, _STEPS)

        def unpack(b, g):
            for j in range(_CHUNK // 16):
                v = pk_v[g, pl.ds(j * 16, 16)]
                src_r[b, pl.ds(j * 16, 16)] = jnp.bitwise_and(v, 0xFFFF)
                dst_r[b, pl.ds(j * 16, 16)] = lax.shift_right_logical(v, 16)

        pltpu.sync_copy(pk_hbm.at[pl.ds(wid * _STEPS, _STEPS)], pk_v)
        unpack(0, 0)
        pltpu.async_copy(x_hbm.at[src_r.at[0]], rows_v.at[0], sems[0])

        zero16 = jnp.zeros((16,), jnp.float32)

        def zrow(i, _):
            for j in range(D_ // 16):
                zbuf[i, pl.ds(j * 16, 16)] = zero16
            return 0

        lax.fori_loop(0, _ZROWS, zrow, 0)
        n_z = _COPY_ROWS // _ZROWS + jnp.where(s == _NS - 1, 1, 0)

        def zfire(j, _):
            pltpu.async_copy(
                zbuf, acc.at[pl.ds(s * _COPY_ROWS + j * _ZROWS, _ZROWS)], zsem
            )
            return 0

        def zdrain(j, _):
            pltpu.make_async_copy(
                zbuf, acc.at[pl.ds(s * _COPY_ROWS, _ZROWS)], zsem
            ).wait()
            return 0

        lax.fori_loop(0, n_z, zfire, 0)
        lax.fori_loop(0, n_z, zdrain, 0)
        plsc.subcore_barrier()

        def step(i, _):
            for b in range(_NBUF):
                g = i * _NBUF + b
                bo = 1 - b

                @pl.when(g + 1 < n_steps)
                def _():
                    @pl.when(g >= 1)
                    def _():
                        pltpu.make_async_copy(
                            rows_v.at[bo], acc.at[dst_r.at[bo]], ssems[bo]
                        ).wait()

                    unpack(bo, g + 1)
                    pltpu.async_copy(
                        x_hbm.at[src_r.at[bo]], rows_v.at[bo], sems[bo]
                    )

                pltpu.make_async_copy(
                    x_hbm.at[src_r.at[b]], rows_v.at[b], sems[b]
                ).wait()
                pltpu.async_copy(
                    rows_v.at[b], acc.at[dst_r.at[b]], ssems[b], add=True
                )

            return 0

        lax.fori_loop(0, (n_steps + _NBUF - 1) // _NBUF, step, 0)
        for b in range(_NBUF):
            pltpu.make_async_copy(
                rows_v.at[b], acc.at[dst_r.at[b]], ssems[b]
            ).wait()
        plsc.subcore_barrier()

        pltpu.sync_copy(
            acc.at[pl.ds(s * _COPY_ROWS, _COPY_ROWS)],
            out_hbm.at[c, pl.ds(s * _COPY_ROWS, _COPY_ROWS)],
        )

        @pl.when(s == _NS - 1)
        def _():
            pltpu.sync_copy(
                acc.at[pl.ds(_NS * _COPY_ROWS, _NACC - _NS * _COPY_ROWS)],
                out_hbm.at[c, pl.ds(_NS * _COPY_ROWS, _NACC - _NS * _COPY_ROWS)],
            )

    return k(x, packed2)


_NB = 10
_BLK = N_NODES_ // _NB


def _mask_kernel(x_ref, ws_ref, mask_ref, p_ref, s_ref):
    i = pl.program_id(0)
    scores = jnp.dot(x_ref[...], ws_ref[...], preferred_element_type=jnp.float32)
    m = scores - jnp.max(scores, axis=1, keepdims=True)
    e = jnp.exp(m)
    nm = e / jnp.sum(e, axis=1, keepdims=True)
    mask_ref[...] = nm
    p_blk = lax.dot_general(
        nm, x_ref[...], (((0,), (0,)), ((), ())), preferred_element_type=jnp.float32
    )
    s_blk = jnp.sum(nm, axis=0, keepdims=True)

    @pl.when(i == 0)
    def _():
        p_ref[...] = p_blk
        s_ref[...] = s_blk

    @pl.when(i > 0)
    def _():
        p_ref[...] += p_blk
        s_ref[...] += s_blk


def _routing(x, w_score):
    return pl.pallas_call(
        _mask_kernel,
        grid=(_NB,),
        in_specs=[
            pl.BlockSpec((_BLK, D_), lambda i: (i, 0)),
            pl.BlockSpec((D_, C_), lambda i: (0, 0)),
        ],
        out_specs=[
            pl.BlockSpec((_BLK, C_), lambda i: (i, 0)),
            pl.BlockSpec((C_, D_), lambda i: (0, 0)),
            pl.BlockSpec((1, C_), lambda i: (0, 0)),
        ],
        out_shape=[
            jax.ShapeDtypeStruct((N_NODES_, C_), jnp.float32),
            jax.ShapeDtypeStruct((C_, D_), jnp.float32),
            jax.ShapeDtypeStruct((1, C_), jnp.float32),
        ],
    )(x, w_score)


def _final_kernel(
    x_ref, mask_ref, a0_ref, a1_ref, p_ref, s_ref,
    wb2c_ref, wbb_ref, wcb_ref, wbc_ref, wcc_ref, wp1_ref, wp2_ref, bp_ref,
    out_ref, accsum,
):
    i = pl.program_id(0)
    denom = s_ref[...].reshape(C_, 1) + 1e-6
    cx = jnp.dot(p_ref[...], wb2c_ref[...], preferred_element_type=jnp.float32) / denom
    mcb = jnp.dot(cx, wcb_ref[...], preferred_element_type=jnp.float32)
    agg = a0_ref[...].astype(jnp.float32) + a1_ref[...].astype(jnp.float32)
    m_bb = jnp.dot(agg, wbb_ref[...], preferred_element_type=jnp.float32)
    m_cb = jnp.dot(mask_ref[...], mcb, preferred_element_type=jnp.float32)
    base = jnp.maximum(m_bb + m_cb + x_ref[...], 0.0)
    bsum = jnp.sum(base, axis=0, keepdims=True)

    @pl.when(i == 0)
    def _():
        accsum[...] = bsum

    @pl.when(i > 0)
    def _():
        accsum[...] += bsum

    @pl.when(i == _NB - 1)
    def _():
        m_bc = jnp.dot(p_ref[...], wbc_ref[...], preferred_element_type=jnp.float32) / denom
        aggc = jnp.dot(
            jnp.sum(cx, axis=0, keepdims=True) - cx,
            wcc_ref[...],
            preferred_element_type=jnp.float32,
        ) / (C_ - 1)
        cent = jnp.maximum(m_bc + aggc + cx, 0.0)
        mean_cent = jnp.mean(cent, axis=0, keepdims=True)
        mean_base = accsum[...] * (1.0 / N_NODES_)
        out_ref[...] = (
            jnp.dot(mean_base, wp1_ref[...], preferred_element_type=jnp.float32)
            + jnp.dot(mean_cent, wp2_ref[...], preferred_element_type=jnp.float32)
            + bp_ref[...]
        )


def _final(x, mask, a0, a1, p, s1, w_base2c, w_bb, w_cb, w_bc, w_cc, wp1, wp2, bp):
    full = lambda shape: pl.BlockSpec(shape, lambda i: tuple(0 for _ in shape))
    return pl.pallas_call(
        _final_kernel,
        grid=(_NB,),
        in_specs=[
            pl.BlockSpec((_BLK, D_), lambda i: (i, 0)),
            pl.BlockSpec((_BLK, C_), lambda i: (i, 0)),
            pl.BlockSpec((_BLK, D_), lambda i: (i, 0)),
            pl.BlockSpec((_BLK, D_), lambda i: (i, 0)),
            full((C_, D_)),
            full((1, C_)),
            full((D_, D_)),
            full((D_, D_)),
            full((D_, D_)),
            full((D_, D_)),
            full((D_, D_)),
            full((D_, D_)),
            full((D_, D_)),
            full((1, D_)),
        ],
        out_specs=pl.BlockSpec((1, D_), lambda i: (0, 0)),
        out_shape=jax.ShapeDtypeStruct((1, D_), jnp.float32),
        scratch_shapes=[pltpu.VMEM((1, D_), jnp.float32)],
    )(x, mask, a0, a1, p, s1, w_base2c, w_bb, w_cb, w_bc, w_cc, wp1, wp2, bp)


def kernel(x, edge_index, W_score, W_base2c, W_bb, W_cb, W_bc, W_cc, W_pred, b_pred):
    packed = edge_index[0] + edge_index[1] * 65536
    packed = jnp.concatenate(
        [packed, jnp.zeros((_EPAD - E_,), jnp.int32)]
    ).reshape(_NW * _STEPS, _CHUNK)
    partials = _edge_partials(x, packed)
    mask, p, s1 = _routing(x, W_score)
    out = _final(
        x, mask, partials[0], partials[1], p, s1,
        W_base2c, W_bb, W_cb, W_bc, W_cc,
        W_pred[:D_], W_pred[D_:], b_pred.reshape(1, D_),
    )
    return out.reshape(D_)


# speedup vs baseline: 1.0236x; 1.0236x over previous
import functools

import jax
import jax.numpy as jnp
from jax import lax
from jax.experimental import pallas as pl
from jax.experimental.pallas import tpu as pltpu
from jax.experimental.pallas import tpu_sc as plsc

N_NODES_ = 10000
D_ = 128
C_ = 8
E_ = 320000

_NC = 2
_NS = 16
_NW = _NC * _NS
_CHUNK = 128
_STEPS = 80
_EPAD = _NW * _STEPS * _CHUNK
_LAST_STEPS = 20
_NACC = N_NODES_
_COPY_ROWS = 624
_ZROWS = 16
_NBUF = 2


def _edge_partials(x, packed2):
    mesh = plsc.VectorSubcoreMesh(core_axis_name="c", subcore_axis_name="s")

    @functools.partial(
        pl.kernel,
        mesh=mesh,
        out_type=jax.ShapeDtypeStruct((_NC, _NACC, D_), jnp.float32),
        scratch_types=[
            pltpu.VMEM((_STEPS, _CHUNK), jnp.int32),
            pltpu.VMEM((_NBUF, _CHUNK), jnp.int32),
            pltpu.VMEM((_NBUF, _CHUNK), jnp.int32),
            pltpu.VMEM((_NBUF, _CHUNK, D_), jnp.float32),
            pltpu.VMEM((_ZROWS, D_), jnp.float32),
            pltpu.VMEM_SHARED((_NACC, D_), jnp.float32),
            pltpu.SemaphoreType.DMA,
            pltpu.SemaphoreType.DMA,
            pltpu.SemaphoreType.DMA,
            pltpu.SemaphoreType.DMA,
            pltpu.SemaphoreType.DMA,
        ],
    )
    def k(x_hbm, pk_hbm, out_hbm, pk_v, src_r, dst_r, rows_v, zbuf, acc,
          sem0, sem1, ss0, ss1, zsem):
        sems = (sem0, sem1)
        ssems = (ss0, ss1)
        c = lax.axis_index("c")
        s = lax.axis_index("s")
        wid = c * _NS + s
        n_steps = jnp.where(wid == _NW - 1, _LAST_STEPS, _STEPS)

        def unpack(b, g):
            for j in range(_CHUNK // 16):
                v = pk_v[g, pl.ds(j * 16, 16)]
                src_r[b, pl.ds(j * 16, 16)] = jnp.bitwise_and(v, 0xFFFF)
                dst_r[b, pl.ds(j * 16, 16)] = lax.shift_right_logical(v, 16)

        pltpu.sync_copy(pk_hbm.at[pl.ds(wid * _STEPS, _STEPS)], pk_v)
        unpack(0, 0)
        pltpu.async_copy(x_hbm.at[src_r.at[0]], rows_v.at[0], sems[0])

        zero16 = jnp.zeros((16,), jnp.float32)

        def zrow(i, _):
            for j in range(D_ // 16):
                zbuf[i, pl.ds(j * 16, 16)] = zero16
            return 0

        lax.fori_loop(0, _ZROWS, zrow, 0)
        n_z = _COPY_ROWS // _ZROWS + jnp.where(s == _NS - 1, 1, 0)

        def zfire(j, _):
            pltpu.async_copy(
                zbuf, acc.at[pl.ds(s * _COPY_ROWS + j * _ZROWS, _ZROWS)], zsem
            )
            return 0

        def zdrain(j, _):
            pltpu.make_async_copy(
                zbuf, acc.at[pl.ds(s * _COPY_ROWS, _ZROWS)], zsem
            ).wait()
            return 0

        lax.fori_loop(0, n_z, zfire, 0)
        lax.fori_loop(0, n_z, zdrain, 0)
        plsc.subcore_barrier()

        def step(i, _):
            for b in range(_NBUF):
                g = i * _NBUF + b
                bo = 1 - b

                @pl.when(g + 1 < n_steps)
                def _():
                    @pl.when(g >= 1)
                    def _():
                        pltpu.make_async_copy(
                            rows_v.at[bo], acc.at[dst_r.at[bo]], ssems[bo]
                        ).wait()

                    unpack(bo, g + 1)
                    pltpu.async_copy(
                        x_hbm.at[src_r.at[bo]], rows_v.at[bo], sems[bo]
                    )

                pltpu.make_async_copy(
                    x_hbm.at[src_r.at[b]], rows_v.at[b], sems[b]
                ).wait()
                pltpu.async_copy(
                    rows_v.at[b], acc.at[dst_r.at[b]], ssems[b], add=True
                )

            return 0

        lax.fori_loop(0, (n_steps + _NBUF - 1) // _NBUF, step, 0)
        for b in range(_NBUF):
            pltpu.make_async_copy(
                rows_v.at[b], acc.at[dst_r.at[b]], ssems[b]
            ).wait()
        plsc.subcore_barrier()

        pltpu.sync_copy(
            acc.at[pl.ds(s * _COPY_ROWS, _COPY_ROWS)],
            out_hbm.at[c, pl.ds(s * _COPY_ROWS, _COPY_ROWS)],
        )

        @pl.when(s == _NS - 1)
        def _():
            pltpu.sync_copy(
                acc.at[pl.ds(_NS * _COPY_ROWS, _NACC - _NS * _COPY_ROWS)],
                out_hbm.at[c, pl.ds(_NS * _COPY_ROWS, _NACC - _NS * _COPY_ROWS)],
            )

    return k(x, packed2)


_NB = 5
_BLK = N_NODES_ // _NB


def _mask_kernel(x_ref, ws_ref, mask_ref, p_ref, s_ref):
    i = pl.program_id(0)
    scores = jnp.dot(x_ref[...], ws_ref[...], preferred_element_type=jnp.float32)
    m = scores - jnp.max(scores, axis=1, keepdims=True)
    e = jnp.exp(m)
    nm = e / jnp.sum(e, axis=1, keepdims=True)
    mask_ref[...] = nm
    p_blk = lax.dot_general(
        nm, x_ref[...], (((0,), (0,)), ((), ())), preferred_element_type=jnp.float32
    )
    s_blk = jnp.sum(nm, axis=0, keepdims=True)

    @pl.when(i == 0)
    def _():
        p_ref[...] = p_blk
        s_ref[...] = s_blk

    @pl.when(i > 0)
    def _():
        p_ref[...] += p_blk
        s_ref[...] += s_blk


def _routing(x, w_score):
    return pl.pallas_call(
        _mask_kernel,
        grid=(_NB,),
        in_specs=[
            pl.BlockSpec((_BLK, D_), lambda i: (i, 0)),
            pl.BlockSpec((D_, C_), lambda i: (0, 0)),
        ],
        out_specs=[
            pl.BlockSpec((_BLK, C_), lambda i: (i, 0)),
            pl.BlockSpec((C_, D_), lambda i: (0, 0)),
            pl.BlockSpec((1, C_), lambda i: (0, 0)),
        ],
        out_shape=[
            jax.ShapeDtypeStruct((N_NODES_, C_), jnp.float32),
            jax.ShapeDtypeStruct((C_, D_), jnp.float32),
            jax.ShapeDtypeStruct((1, C_), jnp.float32),
        ],
    )(x, w_score)


def _final_kernel(
    x_ref, mask_ref, a0_ref, a1_ref, p_ref, s_ref,
    wb2c_ref, wbb_ref, wcb_ref, wbc_ref, wcc_ref, wp1_ref, wp2_ref, bp_ref,
    out_ref, accsum,
):
    i = pl.program_id(0)
    denom = s_ref[...].reshape(C_, 1) + 1e-6
    cx = jnp.dot(p_ref[...], wb2c_ref[...], preferred_element_type=jnp.float32) / denom
    mcb = jnp.dot(cx, wcb_ref[...], preferred_element_type=jnp.float32)
    agg = a0_ref[...].astype(jnp.float32) + a1_ref[...].astype(jnp.float32)
    m_bb = jnp.dot(agg, wbb_ref[...], preferred_element_type=jnp.float32)
    m_cb = jnp.dot(mask_ref[...], mcb, preferred_element_type=jnp.float32)
    base = jnp.maximum(m_bb + m_cb + x_ref[...], 0.0)
    bsum = jnp.sum(base, axis=0, keepdims=True)

    @pl.when(i == 0)
    def _():
        accsum[...] = bsum

    @pl.when(i > 0)
    def _():
        accsum[...] += bsum

    @pl.when(i == _NB - 1)
    def _():
        m_bc = jnp.dot(p_ref[...], wbc_ref[...], preferred_element_type=jnp.float32) / denom
        aggc = jnp.dot(
            jnp.sum(cx, axis=0, keepdims=True) - cx,
            wcc_ref[...],
            preferred_element_type=jnp.float32,
        ) / (C_ - 1)
        cent = jnp.maximum(m_bc + aggc + cx, 0.0)
        mean_cent = jnp.mean(cent, axis=0, keepdims=True)
        mean_base = accsum[...] * (1.0 / N_NODES_)
        out_ref[...] = (
            jnp.dot(mean_base, wp1_ref[...], preferred_element_type=jnp.float32)
            + jnp.dot(mean_cent, wp2_ref[...], preferred_element_type=jnp.float32)
            + bp_ref[...]
        )


def _final(x, mask, a0, a1, p, s1, w_base2c, w_bb, w_cb, w_bc, w_cc, wp1, wp2, bp):
    full = lambda shape: pl.BlockSpec(shape, lambda i: tuple(0 for _ in shape))
    return pl.pallas_call(
        _final_kernel,
        grid=(_NB,),
        in_specs=[
            pl.BlockSpec((_BLK, D_), lambda i: (i, 0)),
            pl.BlockSpec((_BLK, C_), lambda i: (i, 0)),
            pl.BlockSpec((_BLK, D_), lambda i: (i, 0)),
            pl.BlockSpec((_BLK, D_), lambda i: (i, 0)),
            full((C_, D_)),
            full((1, C_)),
            full((D_, D_)),
            full((D_, D_)),
            full((D_, D_)),
            full((D_, D_)),
            full((D_, D_)),
            full((D_, D_)),
            full((D_, D_)),
            full((1, D_)),
        ],
        out_specs=pl.BlockSpec((1, D_), lambda i: (0, 0)),
        out_shape=jax.ShapeDtypeStruct((1, D_), jnp.float32),
        scratch_shapes=[pltpu.VMEM((1, D_), jnp.float32)],
    )(x, mask, a0, a1, p, s1, w_base2c, w_bb, w_cb, w_bc, w_cc, wp1, wp2, bp)


def kernel(x, edge_index, W_score, W_base2c, W_bb, W_cb, W_bc, W_cc, W_pred, b_pred):
    packed = edge_index[0] + edge_index[1] * 65536
    packed = jnp.concatenate(
        [packed, jnp.zeros((_EPAD - E_,), jnp.int32)]
    ).reshape(_NW * _STEPS, _CHUNK)
    partials = _edge_partials(x, packed)
    mask, p, s1 = _routing(x, W_score)
    out = _final(
        x, mask, partials[0], partials[1], p, s1,
        W_base2c, W_bb, W_cb, W_bc, W_cc,
        W_pred[:D_], W_pred[D_:], b_pred.reshape(1, D_),
    )
    return out.reshape(D_)
